# baseline (device time: 17470 ns/iter reference)
import jax
import jax.numpy as jnp
from jax import lax
from jax.experimental import pallas as pl
from jax.experimental.pallas import tpu as pltpu

N_DEV = 32
N_TOK = 256
D_IN = 128
D_OUT = 256
N_EXP = 64
CAP = 3
EXP_PER_DEV = N_EXP // N_DEV
ROWS_PER_DEV = N_TOK // N_DEV
MAX_SEND = EXP_PER_DEV * CAP


def kernel(x, router_W, route_idx, expert_W):
    del router_W

    def body(x_ref, route_ref, w_ref, out_ref,
             comm_ref, counts_ref, tok_ref, surv_ref, nsend_ref,
             send_sems, recv_sems):
        me = lax.axis_index("i")
        my_e_lo = EXP_PER_DEV * me
        my_e_hi = my_e_lo + EXP_PER_DEV
        my_r_lo = ROWS_PER_DEV * me
        my_r_hi = my_r_lo + ROWS_PER_DEV
        barrier_sem = pltpu.get_barrier_semaphore()

        for r in range(ROWS_PER_DEV):
            src = route_ref[my_r_lo + r, 0] // EXP_PER_DEV

            @pl.when(src != me)
            def _():
                pl.semaphore_signal(
                    barrier_sem, inc=1,
                    device_id=(src,),
                    device_id_type=pl.DeviceIdType.MESH,
                )

        def init_counts(j, carry):
            counts_ref[j] = 0
            return carry

        lax.fori_loop(0, N_EXP, init_counts, 0, unroll=8)
        for s in range(MAX_SEND):
            tok_ref[s] = N_TOK
        nsend_ref[0] = 0

        def route_body(i, carry):
            e = route_ref[i, 0]
            c = counts_ref[e]
            under = c < CAP
            counts_ref[e] = c + under.astype(jnp.int32)

            mine = (e >= my_e_lo) & (e < my_e_hi)

            @pl.when(under & mine)
            def _():
                slot = (e - my_e_lo) * CAP + c
                tok_ref[slot] = i

            @pl.when(mine & ((i < my_r_lo) | (i >= my_r_hi)))
            def _():
                nsend_ref[0] = nsend_ref[0] + 1

            @pl.when((i >= my_r_lo) & (i < my_r_hi))
            def _():
                surv_ref[i - my_r_lo] = under.astype(jnp.int32)

            return carry

        lax.fori_loop(0, N_TOK, route_body, 0, unroll=8)

        for r in range(ROWS_PER_DEV):
            @pl.when(surv_ref[r] == 0)
            def _():
                out_ref[pl.ds(r, 1), :] = jnp.zeros((1, D_OUT), jnp.float32)

        toks = [tok_ref[s] for s in range(MAX_SEND)]
        toks_c = [jnp.minimum(t, N_TOK - 1) for t in toks]
        for e_loc in range(EXP_PER_DEV):
            xg = jnp.concatenate(
                [x_ref[pl.ds(toks_c[e_loc * CAP + c], 1), :] for c in range(CAP)],
                axis=0,
            )
            comm_ref[e_loc] = jnp.dot(
                xg.astype(jnp.bfloat16),
                w_ref[e_loc].astype(jnp.bfloat16),
                preferred_element_type=jnp.float32,
            )

        def wait_credit(i, carry):
            pl.semaphore_wait(barrier_sem, 1)
            return carry

        lax.fori_loop(0, nsend_ref[0], wait_credit, 0)

        rdmas = []
        for s in range(MAX_SEND):
            valid = toks[s] < N_TOK
            dst = toks_c[s] // ROWS_PER_DEV
            row = toks_c[s] - dst * ROWS_PER_DEV

            rdma = pltpu.make_async_remote_copy(
                src_ref=comm_ref.at[s // CAP, pl.ds(s % CAP, 1)],
                dst_ref=out_ref.at[pl.ds(row, 1)],
                send_sem=send_sems.at[s],
                recv_sem=recv_sems.at[row],
                device_id=(dst,),
                device_id_type=pl.DeviceIdType.MESH,
            )
            remote = valid & (dst != me)
            rdmas.append((rdma, remote))

            @pl.when(remote)
            def _():
                rdma.start()

            @pl.when(valid & (dst == me))
            def _():
                out_ref[pl.ds(row, 1), :] = comm_ref[s // CAP, pl.ds(s % CAP, 1)]

        for rdma, remote in rdmas:
            @pl.when(remote)
            def _():
                rdma.wait_send()

        for r in range(ROWS_PER_DEV):
            e = route_ref[my_r_lo + r, 0]
            src = e // EXP_PER_DEV
            surv = surv_ref[r] == 1

            @pl.when(surv & (src != me))
            def _():
                recv = pltpu.make_async_remote_copy(
                    src_ref=comm_ref.at[0, pl.ds(0, 1)],
                    dst_ref=out_ref.at[pl.ds(r, 1)],
                    send_sem=send_sems.at[0],
                    recv_sem=recv_sems.at[r],
                    device_id=(me,),
                    device_id_type=pl.DeviceIdType.MESH,
                )
                recv.wait_recv()

    return pl.pallas_call(
        body,
        out_shape=jax.ShapeDtypeStruct((ROWS_PER_DEV, D_OUT), jnp.float32),
        in_specs=[
            pl.BlockSpec(memory_space=pltpu.VMEM),
            pl.BlockSpec(memory_space=pltpu.SMEM),
            pl.BlockSpec(memory_space=pltpu.VMEM),
        ],
        out_specs=pl.BlockSpec(memory_space=pltpu.VMEM),
        scratch_shapes=[
            pltpu.VMEM((EXP_PER_DEV, CAP, D_OUT), jnp.float32),
            pltpu.SMEM((N_EXP,), jnp.int32),
            pltpu.SMEM((MAX_SEND,), jnp.int32),
            pltpu.SMEM((ROWS_PER_DEV,), jnp.int32),
            pltpu.SMEM((1,), jnp.int32),
            pltpu.SemaphoreType.DMA((MAX_SEND,)),
            pltpu.SemaphoreType.DMA((ROWS_PER_DEV,)),
        ],
        compiler_params=pltpu.CompilerParams(collective_id=0),
    )(x, route_idx, expert_W)


# device time: 15903 ns/iter; 1.0985x vs baseline; 1.0985x over previous
import jax
import jax.numpy as jnp
from jax import lax
from jax.experimental import pallas as pl
from jax.experimental.pallas import tpu as pltpu

N_DEV = 32
N_TOK = 256
D_IN = 128
D_OUT = 256
N_EXP = 64
CAP = 3
EXP_PER_DEV = N_EXP // N_DEV
ROWS_PER_DEV = N_TOK // N_DEV
MAX_SEND = EXP_PER_DEV * CAP


def kernel(x, router_W, route_idx, expert_W):
    del router_W

    def body(x_ref, route_ref, w_ref, out_ref,
             comm_ref, counts_ref, tok_ref, surv_ref, nsend_ref,
             send_sems, recv_sems):
        me = lax.axis_index("i")
        my_e_lo = EXP_PER_DEV * me
        my_e_hi = my_e_lo + EXP_PER_DEV
        my_r_lo = ROWS_PER_DEV * me
        my_r_hi = my_r_lo + ROWS_PER_DEV
        barrier_sem = pltpu.get_barrier_semaphore()

        for r in range(ROWS_PER_DEV):
            src = route_ref[my_r_lo + r, 0] // EXP_PER_DEV

            @pl.when(src != me)
            def _():
                pl.semaphore_signal(
                    barrier_sem, inc=1,
                    device_id=(src,),
                    device_id_type=pl.DeviceIdType.MESH,
                )

        def init_counts(j, carry):
            counts_ref[j] = 0
            return carry

        lax.fori_loop(0, N_EXP, init_counts, 0, unroll=8)
        for s in range(MAX_SEND):
            tok_ref[s] = N_TOK
        nsend_ref[0] = 0

        def route_body(i, carry):
            e = route_ref[i, 0]
            c = counts_ref[e]
            under = c < CAP
            counts_ref[e] = c + under.astype(jnp.int32)

            mine = (e >= my_e_lo) & (e < my_e_hi)

            @pl.when(under & mine)
            def _():
                slot = (e - my_e_lo) * CAP + c
                tok_ref[slot] = i

            @pl.when(mine & ((i < my_r_lo) | (i >= my_r_hi)))
            def _():
                nsend_ref[0] = nsend_ref[0] + 1

            @pl.when((i >= my_r_lo) & (i < my_r_hi))
            def _():
                surv_ref[i - my_r_lo] = under.astype(jnp.int32)

            return carry

        lax.fori_loop(0, N_TOK, route_body, 0, unroll=4)

        for r in range(ROWS_PER_DEV):
            @pl.when(surv_ref[r] == 0)
            def _():
                out_ref[pl.ds(r, 1), :] = jnp.zeros((1, D_OUT), jnp.float32)

        toks = [tok_ref[s] for s in range(MAX_SEND)]
        toks_c = [jnp.minimum(t, N_TOK - 1) for t in toks]
        for e_loc in range(EXP_PER_DEV):
            xg = jnp.concatenate(
                [x_ref[pl.ds(toks_c[e_loc * CAP + c], 1), :] for c in range(CAP)],
                axis=0,
            )
            comm_ref[e_loc] = jnp.dot(
                xg.astype(jnp.bfloat16),
                w_ref[e_loc].astype(jnp.bfloat16),
                preferred_element_type=jnp.float32,
            )

        def wait_credit(i, carry):
            pl.semaphore_wait(barrier_sem, 1)
            return carry

        lax.fori_loop(0, nsend_ref[0], wait_credit, 0)

        rdmas = []
        for s in range(MAX_SEND):
            valid = toks[s] < N_TOK
            dst = toks_c[s] // ROWS_PER_DEV
            row = toks_c[s] - dst * ROWS_PER_DEV

            rdma = pltpu.make_async_remote_copy(
                src_ref=comm_ref.at[s // CAP, pl.ds(s % CAP, 1)],
                dst_ref=out_ref.at[pl.ds(row, 1)],
                send_sem=send_sems.at[s],
                recv_sem=recv_sems.at[row],
                device_id=(dst,),
                device_id_type=pl.DeviceIdType.MESH,
            )
            remote = valid & (dst != me)
            rdmas.append((rdma, remote))

            @pl.when(remote)
            def _():
                rdma.start()

            @pl.when(valid & (dst == me))
            def _():
                out_ref[pl.ds(row, 1), :] = comm_ref[s // CAP, pl.ds(s % CAP, 1)]

        for rdma, remote in rdmas:
            @pl.when(remote)
            def _():
                rdma.wait_send()

        for r in range(ROWS_PER_DEV):
            e = route_ref[my_r_lo + r, 0]
            src = e // EXP_PER_DEV
            surv = surv_ref[r] == 1

            @pl.when(surv & (src != me))
            def _():
                recv = pltpu.make_async_remote_copy(
                    src_ref=comm_ref.at[0, pl.ds(0, 1)],
                    dst_ref=out_ref.at[pl.ds(r, 1)],
                    send_sem=send_sems.at[0],
                    recv_sem=recv_sems.at[r],
                    device_id=(me,),
                    device_id_type=pl.DeviceIdType.MESH,
                )
                recv.wait_recv()

    return pl.pallas_call(
        body,
        out_shape=jax.ShapeDtypeStruct((ROWS_PER_DEV, D_OUT), jnp.float32),
        in_specs=[
            pl.BlockSpec(memory_space=pltpu.VMEM),
            pl.BlockSpec(memory_space=pltpu.SMEM),
            pl.BlockSpec(memory_space=pltpu.VMEM),
        ],
        out_specs=pl.BlockSpec(memory_space=pltpu.VMEM),
        scratch_shapes=[
            pltpu.VMEM((EXP_PER_DEV, CAP, D_OUT), jnp.float32),
            pltpu.SMEM((N_EXP,), jnp.int32),
            pltpu.SMEM((MAX_SEND,), jnp.int32),
            pltpu.SMEM((ROWS_PER_DEV,), jnp.int32),
            pltpu.SMEM((1,), jnp.int32),
            pltpu.SemaphoreType.DMA((MAX_SEND,)),
            pltpu.SemaphoreType.DMA((ROWS_PER_DEV,)),
        ],
        compiler_params=pltpu.CompilerParams(collective_id=0),
    )(x, route_idx, expert_W)
